# baseline (speedup 1.0000x reference)
import functools

import jax
import jax.numpy as jnp
from jax.experimental import pallas as pl
from jax.experimental.pallas import tpu as pltpu

_MIN_DISABLED = 32
_MAX_DISABLED = 128
_LANE = 128
_N_BUF = 4
_CHUNK_SIZES = [256, 256, 512] + [1024] * 14 + [512, 256, 256]
_MAX_CHUNK = max(_CHUNK_SIZES)


@functools.cache
def _disabled_span(tof_count: int) -> tuple[int, int]:
    with jax.ensure_compile_time_eval():
        key = jax.random.key(42)
        k1, k2 = jax.random.split(key)
        count = int(jax.random.randint(k1, (), _MIN_DISABLED, _MAX_DISABLED + 1))
        start = int(jax.random.randint(k2, (), 0, tof_count))
    return start, count


def _bounce_body(img_ref, out_ref, *rest, masked_tiles, start, count,
                 tof_count):
    n_chunks = len(_CHUNK_SIZES)
    offsets = [sum(_CHUNK_SIZES[:c]) for c in range(n_chunks)]
    bufs = rest[:_N_BUF]
    rsems = rest[_N_BUF:_N_BUF + n_chunks]
    wsems = rest[_N_BUF + n_chunks:]

    def read_cp(c):
        o, s = offsets[c], _CHUNK_SIZES[c]
        return pltpu.make_async_copy(
            img_ref.at[o:o + s], bufs[c % _N_BUF].at[0:s], rsems[c])

    def write_cp(c):
        o, s = offsets[c], _CHUNK_SIZES[c]
        return pltpu.make_async_copy(
            bufs[c % _N_BUF].at[0:s], out_ref.at[o:o + s], wsems[c])

    reads = {}
    writes = {}
    reads[0] = read_cp(0)
    reads[0].start()
    for c in range(n_chunks):
        nxt = c + 1
        if nxt < n_chunks:
            if nxt >= _N_BUF:
                writes[nxt - _N_BUF].wait()
            reads[nxt] = read_cp(nxt)
            reads[nxt].start()
        reads[c].wait()
        buf = bufs[c % _N_BUF]
        s = _CHUNK_SIZES[c]
        for t in masked_tiles:
            strip = buf[0:s, t * _LANE:(t + 1) * _LANE]
            cols = t * _LANE + jax.lax.broadcasted_iota(
                jnp.int32, strip.shape, 1)
            disabled = ((cols - start) % tof_count) < count
            buf[0:s, t * _LANE:(t + 1) * _LANE] = jnp.where(
                disabled, jnp.float32(0.0), strip)
        writes[c] = write_cp(c)
        writes[c].start()
    for c in range(max(0, n_chunks - _N_BUF), n_chunks):
        writes[c].wait()


def kernel(img):
    rows, tof_count = img.shape
    assert rows == sum(_CHUNK_SIZES)
    start, count = _disabled_span(tof_count)
    end = start + count

    n_tiles = tof_count // _LANE
    t0 = start // _LANE
    t1 = ((end - 1) // _LANE) % n_tiles
    masked_tiles = sorted({t0, t1})

    n_chunks = len(_CHUNK_SIZES)
    body = functools.partial(
        _bounce_body, masked_tiles=masked_tiles,
        start=start, count=count, tof_count=tof_count)
    return pl.pallas_call(
        body,
        in_specs=[pl.BlockSpec(memory_space=pl.ANY)],
        out_specs=pl.BlockSpec(memory_space=pl.ANY),
        out_shape=jax.ShapeDtypeStruct((rows, tof_count), jnp.float32),
        scratch_shapes=(
            [pltpu.VMEM((_MAX_CHUNK, tof_count), jnp.float32)
             for _ in range(_N_BUF)]
            + [pltpu.SemaphoreType.DMA for _ in range(2 * n_chunks)]
        ),
    )(img)


# speedup vs baseline: 1.0044x; 1.0044x over previous
import functools

import jax
import jax.numpy as jnp
from jax.experimental import pallas as pl
from jax.experimental.pallas import tpu as pltpu

_MIN_DISABLED = 32
_MAX_DISABLED = 128
_LANE = 128
_N_CHUNKS = 16
_N_BUF = 6


@functools.cache
def _disabled_span(tof_count: int) -> tuple[int, int]:
    with jax.ensure_compile_time_eval():
        key = jax.random.key(42)
        k1, k2 = jax.random.split(key)
        count = int(jax.random.randint(k1, (), _MIN_DISABLED, _MAX_DISABLED + 1))
        start = int(jax.random.randint(k2, (), 0, tof_count))
    return start, count


def _bounce_body(img_ref, out_ref, *rest, masked_tiles, start, count,
                 tof_count):
    bufs = rest[:_N_BUF]
    rsems = rest[_N_BUF:_N_BUF + _N_CHUNKS]
    wsems = rest[_N_BUF + _N_CHUNKS:]
    rows = img_ref.shape[0]
    chunk_rows = rows // _N_CHUNKS

    def read_cp(c):
        return pltpu.make_async_copy(
            img_ref.at[c * chunk_rows:(c + 1) * chunk_rows],
            bufs[c % _N_BUF], rsems[c])

    def write_cp(c):
        return pltpu.make_async_copy(
            bufs[c % _N_BUF],
            out_ref.at[c * chunk_rows:(c + 1) * chunk_rows], wsems[c])

    reads = {}
    writes = {}
    reads[0] = read_cp(0)
    reads[0].start()
    for c in range(_N_CHUNKS):
        nxt = c + 1
        if nxt < _N_CHUNKS:
            if nxt >= _N_BUF:
                writes[nxt - _N_BUF].wait()
            reads[nxt] = read_cp(nxt)
            reads[nxt].start()
        reads[c].wait()
        buf = bufs[c % _N_BUF]
        for t in masked_tiles:
            strip = buf[:, t * _LANE:(t + 1) * _LANE]
            cols = t * _LANE + jax.lax.broadcasted_iota(
                jnp.int32, strip.shape, 1)
            disabled = ((cols - start) % tof_count) < count
            buf[:, t * _LANE:(t + 1) * _LANE] = jnp.where(
                disabled, jnp.float32(0.0), strip)
        writes[c] = write_cp(c)
        writes[c].start()
    for c in range(max(0, _N_CHUNKS - _N_BUF), _N_CHUNKS):
        writes[c].wait()


def kernel(img):
    rows, tof_count = img.shape
    start, count = _disabled_span(tof_count)
    end = start + count

    n_tiles = tof_count // _LANE
    t0 = start // _LANE
    t1 = ((end - 1) // _LANE) % n_tiles
    masked_tiles = sorted({t0, t1})

    chunk_rows = rows // _N_CHUNKS
    body = functools.partial(
        _bounce_body, masked_tiles=masked_tiles,
        start=start, count=count, tof_count=tof_count)
    return pl.pallas_call(
        body,
        in_specs=[pl.BlockSpec(memory_space=pl.ANY)],
        out_specs=pl.BlockSpec(memory_space=pl.ANY),
        out_shape=jax.ShapeDtypeStruct((rows, tof_count), jnp.float32),
        scratch_shapes=(
            [pltpu.VMEM((chunk_rows, tof_count), jnp.float32)
             for _ in range(_N_BUF)]
            + [pltpu.SemaphoreType.DMA for _ in range(2 * _N_CHUNKS)]
        ),
    )(img)
